# Initial kernel scaffold; baseline (speedup 1.0000x reference)
import jax
import jax.numpy as jnp
from jax.experimental import pallas as pl


def kernel(src, tgt, emb_table, pos_src_table, pos_tgt_table):
    raise NotImplementedError("write your pallas kernel here")


# speedup vs baseline: 1.9283x; 1.9283x over previous
import functools

import numpy as np
import jax
import jax.numpy as jnp
from jax import lax
from jax.experimental import pallas as pl
from jax.experimental.pallas import tpu as pltpu
from jax.experimental.pallas import tpu_sc as plsc

D = 512
B = 256
L = 512
NTOK = B * L
SCALE = float(np.sqrt(D))

_info = plsc.get_sparse_core_info()
NC = _info.num_cores
NS = _info.num_subcores
LANES = _info.num_lanes
NW = NC * NS
TOK_PER_W = NTOK // NW
ROWS_PER_W = TOK_PER_W // L
C = 64
NCHUNK = L // C

_mesh = plsc.VectorSubcoreMesh(core_axis_name="c", subcore_axis_name="s")


@functools.partial(
    pl.kernel,
    mesh=_mesh,
    out_type=jax.ShapeDtypeStruct((2 * NTOK, D), jnp.float32),
    scratch_types=[
        pltpu.VMEM((C,), jnp.int32),
        pltpu.VMEM((C, D), jnp.float32),
        pltpu.VMEM((C, D), jnp.float32),
        pltpu.SemaphoreType.DMA,
    ],
)
def _embed_sc(src_hbm, tgt_hbm, emb_hbm, pos_src_hbm, pos_tgt_hbm, out_hbm,
              idx_v, rows_v, pos_v, sem):
    wid = lax.axis_index("s") * NC + lax.axis_index("c")
    for side in range(2):
        idx_hbm = src_hbm if side == 0 else tgt_hbm
        pos_hbm = pos_src_hbm if side == 0 else pos_tgt_hbm

        def c_body(c, _):
            pltpu.sync_copy(pos_hbm.at[pl.ds(c * C, C)], pos_v)

            def r_body(r, _):
                tok0 = wid * TOK_PER_W + r * L + c * C
                pltpu.sync_copy(idx_hbm.at[pl.ds(tok0, C)], idx_v)
                pltpu.async_copy(emb_hbm.at[idx_v], rows_v, sem).wait()

                def i_body(i, _):
                    for j in range(D // LANES):
                        sl = pl.ds(j * LANES, LANES)
                        rows_v[i, sl] = rows_v[i, sl] * SCALE + pos_v[i, sl]
                    return 0

                lax.fori_loop(0, C, i_body, 0)
                pltpu.sync_copy(rows_v, out_hbm.at[pl.ds(side * NTOK + tok0, C)])
                return 0

            lax.fori_loop(0, ROWS_PER_W, r_body, 0)
            return 0

        lax.fori_loop(0, NCHUNK, c_body, 0)


def kernel(src, tgt, emb_table, pos_src_table, pos_tgt_table):
    out = _embed_sc(src.reshape(-1), tgt.reshape(-1), emb_table,
                    pos_src_table, pos_tgt_table)
    return out.reshape(2, B, L, D)


# speedup vs baseline: 2.8820x; 1.4946x over previous
import functools

import numpy as np
import jax
import jax.numpy as jnp
from jax import lax
from jax.experimental import pallas as pl
from jax.experimental.pallas import tpu as pltpu
from jax.experimental.pallas import tpu_sc as plsc

D = 512
B = 256
L = 512
NTOK = B * L
SCALE = float(np.sqrt(D))

_info = plsc.get_sparse_core_info()
NC = _info.num_cores
NS = _info.num_subcores
LANES = _info.num_lanes
NW = NC * NS
TOK_PER_W = NTOK // NW
ROWS_PER_W = TOK_PER_W // L
C = 64
NCHUNK = L // C

_mesh = plsc.VectorSubcoreMesh(core_axis_name="c", subcore_axis_name="s")


@functools.partial(
    pl.kernel,
    mesh=_mesh,
    out_type=jax.ShapeDtypeStruct((2 * NTOK, D), jnp.float32),
    scratch_types=[
        pltpu.VMEM((TOK_PER_W,), jnp.int32),
        pltpu.VMEM((C, D), jnp.float32),
        pltpu.VMEM((C, D), jnp.float32),
        pltpu.VMEM((C, D), jnp.float32),
        pltpu.SemaphoreType.DMA,
        pltpu.SemaphoreType.DMA,
        pltpu.SemaphoreType.DMA,
        pltpu.SemaphoreType.DMA,
    ],
)
def _embed_sc(src_hbm, tgt_hbm, emb_hbm, pos_src_hbm, pos_tgt_hbm, out_hbm,
              idx_v, rows0_v, rows1_v, pos_v, g0, g1, s0, s1):
    wid = lax.axis_index("s") * NC + lax.axis_index("c")
    rows = (rows0_v, rows1_v)
    gsem = (g0, g1)
    ssem = (s0, s1)

    def fma(buf):
        def i_body(i, _):
            for j in range(D // LANES):
                sl = pl.ds(j * LANES, LANES)
                buf[i, sl] = buf[i, sl] * SCALE + pos_v[i, sl]
            return 0

        lax.fori_loop(0, C, i_body, 0)

    for side in range(2):
        idx_hbm = src_hbm if side == 0 else tgt_hbm
        pos_hbm = pos_src_hbm if side == 0 else pos_tgt_hbm
        pltpu.sync_copy(idx_hbm.at[pl.ds(wid * TOK_PER_W, TOK_PER_W)], idx_v)

        def c_body(c, _):
            pltpu.sync_copy(pos_hbm.at[pl.ds(c * C, C)], pos_v)

            def gather(r, b):
                off = r * L + c * C
                pltpu.async_copy(emb_hbm.at[idx_v.at[pl.ds(off, C)]],
                                 rows[b], gsem[b])

            def scatter(r, b):
                tok0 = side * NTOK + wid * TOK_PER_W + r * L + c * C
                pltpu.async_copy(rows[b], out_hbm.at[pl.ds(tok0, C)], ssem[b])

            gather(0, 0)
            for r in range(ROWS_PER_W):
                b = r % 2
                nb = (r + 1) % 2
                if r + 1 < ROWS_PER_W:
                    if r >= 1:
                        pltpu.make_async_copy(rows[nb],
                                              out_hbm.at[pl.ds(0, C)],
                                              ssem[nb]).wait()
                    gather(r + 1, nb)
                pltpu.make_async_copy(emb_hbm.at[idx_v.at[pl.ds(0, C)]],
                                      rows[b], gsem[b]).wait()
                fma(rows[b])
                scatter(r, b)
            pltpu.make_async_copy(rows[0], out_hbm.at[pl.ds(0, C)], ssem[0]).wait()
            pltpu.make_async_copy(rows[1], out_hbm.at[pl.ds(0, C)], ssem[1]).wait()
            return 0

        lax.fori_loop(0, NCHUNK, c_body, 0)


def kernel(src, tgt, emb_table, pos_src_table, pos_tgt_table):
    out = _embed_sc(src.reshape(-1), tgt.reshape(-1), emb_table,
                    pos_src_table, pos_tgt_table)
    return out.reshape(2, B, L, D)
